# baseline (speedup 1.0000x reference)
import functools

import jax
import jax.numpy as jnp
from jax import lax
from jax.experimental import pallas as pl
from jax.experimental.pallas import tpu as pltpu
from jax.experimental.pallas import tpu_sc as plsc

B = 8192
K = 8192
D_IN = 512
D_E = 32

TB = 256
NT = B // TB

DB = 1024
ND = B // DB


def _encode_dist_body(x_ref, w1_ref, b1_ref, w2_ref, b2_ref, cwt_ref,
                      idx_ref, dist_ref):
    x = x_ref[...]
    h = jnp.maximum(jnp.dot(x, w1_ref[...]) + b1_ref[...], 0.0)
    e = jnp.maximum(jnp.dot(h, w2_ref[...]) + b2_ref[...], 0.0)
    cwt = cwt_ref[...]
    c2 = jnp.sum(cwt * cwt, axis=0, keepdims=True)
    a2 = jnp.sum(e * e, axis=1, keepdims=True)
    d2 = jnp.maximum(a2 + c2 - jnp.dot(e + e, cwt), 0.0)
    mn2 = jnp.min(d2, axis=1, keepdims=True)
    s = jnp.sqrt(mn2)
    u = mn2
    iota = lax.broadcasted_iota(jnp.int32, d2.shape, 1)
    idx = jnp.min(jnp.where(d2 <= u, iota, K), axis=1)
    idx_ref[...] = idx.reshape(1, 1, TB)
    dist_ref[...] = s.reshape(1, 1, TB)


def _decode_body(q_ref, w1_ref, b1_ref, w2_ref, b2_ref, out_ref):
    q = q_ref[...][:, :D_E]
    d = jnp.maximum(jnp.dot(q, w1_ref[...]) + b1_ref[...], 0.0)
    out_ref[...] = jnp.dot(d, w2_ref[...]) + b2_ref[...]


_SC_CORES = 2
_SC_SUBCORES = 16
_NW = _SC_CORES * _SC_SUBCORES
_BPW = B // _NW
D_PAD = 128


_NCHUNK = 8
_CROWS = _BPW // _NCHUNK


@functools.cache
def _make_sc_gather():
    @functools.partial(
        pl.kernel,
        mesh=plsc.VectorSubcoreMesh(core_axis_name="c", subcore_axis_name="s"),
        out_type=jax.ShapeDtypeStruct((B, D_PAD), jnp.float32),
        scratch_types=[
            pltpu.VMEM((_BPW,), jnp.int32),
            pltpu.VMEM((_BPW, D_PAD), jnp.float32),
            pltpu.SemaphoreType.DMA,
        ],
    )
    def _sc_gather(table_hbm, idx_hbm, out_hbm, idx_v, rows_v, sem):
        wid = lax.axis_index("s") * _SC_CORES + lax.axis_index("c")
        base = wid * _BPW
        pltpu.sync_copy(idx_hbm.at[pl.ds(base, _BPW)], idx_v)
        copies = [
            pltpu.async_copy(
                table_hbm.at[idx_v.at[pl.ds(t * _CROWS, _CROWS)]],
                rows_v.at[pl.ds(t * _CROWS, _CROWS)],
                sem,
            )
            for t in range(_NCHUNK)
        ]
        for c in copies:
            c.wait()
        pltpu.sync_copy(rows_v, out_hbm.at[pl.ds(base, _BPW)])

    return _sc_gather


def kernel(x, W1e, b1e, W2e, b2e, codewords, W1d, b1d, W2d, b2d):
    idx3, dist3 = pl.pallas_call(
        _encode_dist_body,
        grid=(NT,),
        in_specs=[
            pl.BlockSpec((TB, D_IN), lambda i: (i, 0)),
            pl.BlockSpec((D_IN, D_E), lambda i: (0, 0)),
            pl.BlockSpec((1, D_E), lambda i: (0, 0)),
            pl.BlockSpec((D_E, D_E), lambda i: (0, 0)),
            pl.BlockSpec((1, D_E), lambda i: (0, 0)),
            pl.BlockSpec((D_E, K), lambda i: (0, 0)),
        ],
        out_specs=[
            pl.BlockSpec((1, 1, TB), lambda i: (i, 0, 0)),
            pl.BlockSpec((1, 1, TB), lambda i: (i, 0, 0)),
        ],
        out_shape=[
            jax.ShapeDtypeStruct((NT, 1, TB), jnp.int32),
            jax.ShapeDtypeStruct((NT, 1, TB), jnp.float32),
        ],
    )(x, W1e.T, b1e.reshape(1, D_E), W2e.T, b2e.reshape(1, D_E),
      codewords.T)

    quantized_indices = idx3.reshape(B)
    quantized_distances = dist3.reshape(B)

    cw_pad = jnp.pad(codewords, ((0, 0), (0, D_PAD - D_E)))
    rows128 = _make_sc_gather()(cw_pad, quantized_indices)
    quantized = rows128[:, :D_E]

    reconstructed = pl.pallas_call(
        _decode_body,
        grid=(ND,),
        in_specs=[
            pl.BlockSpec((DB, D_PAD), lambda i: (i, 0)),
            pl.BlockSpec((D_E, D_E), lambda i: (0, 0)),
            pl.BlockSpec((1, D_E), lambda i: (0, 0)),
            pl.BlockSpec((D_E, D_IN), lambda i: (0, 0)),
            pl.BlockSpec((1, D_IN), lambda i: (0, 0)),
        ],
        out_specs=pl.BlockSpec((DB, D_IN), lambda i: (i, 0)),
        out_shape=jax.ShapeDtypeStruct((B, D_IN), jnp.float32),
    )(quantized, W1d.T, b1d.reshape(1, D_E), W2d.T, b2d.reshape(1, D_IN))

    return (quantized_indices, quantized_distances, reconstructed, quantized)


# speedup vs baseline: 2.0848x; 2.0848x over previous
import functools

import jax
import jax.numpy as jnp
from jax import lax
from jax.experimental import pallas as pl
from jax.experimental.pallas import tpu as pltpu
from jax.experimental.pallas import tpu_sc as plsc

B = 8192
K = 8192
D_IN = 512
D_E = 32

TB = 256
NT = B // TB

DB = 1024
ND = B // DB


def _encode_dist_body(x_ref, w1_ref, b1_ref, w2_ref, b2_ref, cwt_ref,
                      idx_ref, dist_ref):
    x = x_ref[...]
    h = jnp.maximum(jnp.dot(x, w1_ref[...]) + b1_ref[...], 0.0)
    e = jnp.maximum(jnp.dot(h, w2_ref[...]) + b2_ref[...], 0.0)
    cwt = cwt_ref[...]
    c2 = jnp.sum(cwt * cwt, axis=0, keepdims=True)
    a2 = jnp.sum(e * e, axis=1, keepdims=True)
    d2 = jnp.maximum(a2 + c2 - jnp.dot(e + e, cwt), 0.0)
    mn2 = jnp.min(d2, axis=1, keepdims=True)
    s = jnp.sqrt(mn2)
    u = mn2
    iota = lax.broadcasted_iota(jnp.int32, d2.shape, 1)
    idx = jnp.min(jnp.where(d2 <= u, iota, K), axis=1)
    idx_ref[...] = idx.reshape(1, 1, TB)
    dist_ref[...] = s.reshape(1, 1, TB)


def _decode_body(q_ref, w1_ref, b1_ref, w2_ref, b2_ref, out_ref):
    q = q_ref[...][:, :D_E]
    d = jnp.maximum(jnp.dot(q, w1_ref[...]) + b1_ref[...], 0.0)
    out_ref[...] = jnp.dot(d, w2_ref[...]) + b2_ref[...]


_SC_CORES = 2
_SC_SUBCORES = 16
_NW = _SC_CORES * _SC_SUBCORES
_BPW = B // _NW
D_PAD = 128


_NCHUNK = 8
_CROWS = _BPW // _NCHUNK


@functools.cache
def _make_sc_gather():
    @functools.partial(
        pl.kernel,
        mesh=plsc.VectorSubcoreMesh(core_axis_name="c", subcore_axis_name="s"),
        out_type=jax.ShapeDtypeStruct((B, D_PAD), jnp.float32),
        scratch_types=[
            pltpu.VMEM((_BPW,), jnp.int32),
            pltpu.VMEM((_BPW, D_PAD), jnp.float32),
            pltpu.VMEM_SHARED((K, D_PAD), jnp.float32),
            pltpu.SemaphoreType.DMA,
        ],
    )
    def _sc_gather(table_hbm, idx_hbm, out_hbm, idx_v, rows_v, cw_sh, sem):
        sid = lax.axis_index("s")
        wid = sid * _SC_CORES + lax.axis_index("c")
        base = wid * _BPW
        pltpu.sync_copy(idx_hbm.at[pl.ds(base, _BPW)], idx_v)
        @pl.when(sid == 0)
        def _():
            pltpu.sync_copy(table_hbm, cw_sh)
        plsc.subcore_barrier()
        copies = [
            pltpu.async_copy(
                cw_sh.at[idx_v.at[pl.ds(t * _CROWS, _CROWS)]],
                rows_v.at[pl.ds(t * _CROWS, _CROWS)],
                sem,
            )
            for t in range(_NCHUNK)
        ]
        for c in copies:
            c.wait()
        pltpu.sync_copy(rows_v, out_hbm.at[pl.ds(base, _BPW)])

    return _sc_gather


def kernel(x, W1e, b1e, W2e, b2e, codewords, W1d, b1d, W2d, b2d):
    idx3, dist3 = pl.pallas_call(
        _encode_dist_body,
        grid=(NT,),
        in_specs=[
            pl.BlockSpec((TB, D_IN), lambda i: (i, 0)),
            pl.BlockSpec((D_IN, D_E), lambda i: (0, 0)),
            pl.BlockSpec((1, D_E), lambda i: (0, 0)),
            pl.BlockSpec((D_E, D_E), lambda i: (0, 0)),
            pl.BlockSpec((1, D_E), lambda i: (0, 0)),
            pl.BlockSpec((D_E, K), lambda i: (0, 0)),
        ],
        out_specs=[
            pl.BlockSpec((1, 1, TB), lambda i: (i, 0, 0)),
            pl.BlockSpec((1, 1, TB), lambda i: (i, 0, 0)),
        ],
        out_shape=[
            jax.ShapeDtypeStruct((NT, 1, TB), jnp.int32),
            jax.ShapeDtypeStruct((NT, 1, TB), jnp.float32),
        ],
    )(x, W1e.T, b1e.reshape(1, D_E), W2e.T, b2e.reshape(1, D_E),
      codewords.T)

    quantized_indices = idx3.reshape(B)
    quantized_distances = dist3.reshape(B)

    cw_pad = jnp.pad(codewords, ((0, 0), (0, D_PAD - D_E)))
    rows128 = _make_sc_gather()(cw_pad, quantized_indices)
    quantized = rows128[:, :D_E]

    reconstructed = pl.pallas_call(
        _decode_body,
        grid=(ND,),
        in_specs=[
            pl.BlockSpec((DB, D_PAD), lambda i: (i, 0)),
            pl.BlockSpec((D_E, D_E), lambda i: (0, 0)),
            pl.BlockSpec((1, D_E), lambda i: (0, 0)),
            pl.BlockSpec((D_E, D_IN), lambda i: (0, 0)),
            pl.BlockSpec((1, D_IN), lambda i: (0, 0)),
        ],
        out_specs=pl.BlockSpec((DB, D_IN), lambda i: (i, 0)),
        out_shape=jax.ShapeDtypeStruct((B, D_IN), jnp.float32),
    )(quantized, W1d.T, b1d.reshape(1, D_E), W2d.T, b2d.reshape(1, D_IN))

    return (quantized_indices, quantized_distances, reconstructed, quantized)
